# baseline (speedup 1.0000x reference)
import jax
import jax.numpy as jnp
from jax import lax
from jax.experimental import pallas as pl
from jax.experimental.pallas import tpu as pltpu
from jax.experimental.pallas import tpu_sc as plsc

B = 4
S = 8192
D = 128
N = B * S

NC = 2
NS = 16
NW = NC * NS

PPW = S // NW
CH = 64
SUBS = PPW // CH
NCH = B * SUBS
NBUF = 10
LA = 7
LANES = 16
VPR = D // LANES


def _body(x_hbm, tok_hbm, pos_hbm, out_hbm, idx_v, pos_v, bufs, isem, gsem, wsem):
    c = lax.axis_index("c")
    s = lax.axis_index("s")
    wid = s * NC + c
    p0 = wid * PPW

    idx_copies = [
        pltpu.async_copy(
            x_hbm.at[bt, pl.ds(p0 + sub * CH, CH)],
            idx_v.at[bt * SUBS + sub],
            isem,
        )
        for bt in range(B)
        for sub in range(SUBS)
    ]
    pltpu.sync_copy(pos_hbm.at[pl.ds(p0, PPW)], pos_v)
    for cp in idx_copies:
        cp.wait()


    gathers, writebacks = {}, {}

    def prefill_and_gather(t):
        if t < NCH:
            if t >= NBUF:
                writebacks.pop(t - NBUF).wait()
            buf = bufs.at[t % NBUF]
            prow = (t % SUBS) * CH

            def copy_row(r, carry):
                for k in range(VPR):
                    sl = pl.ds(k * LANES, LANES)
                    buf[r, sl] = pos_v[prow + r, sl]
                return carry

            lax.fori_loop(0, CH, copy_row, 0)
            gathers[t] = pltpu.async_copy(
                tok_hbm.at[idx_v.at[t]], buf, gsem, add=True)

    for t in range(LA):
        prefill_and_gather(t)

    for j in range(NCH):
        prefill_and_gather(j + LA)
        gathers.pop(j).wait()
        bt, sub = j // SUBS, j % SUBS
        writebacks[j] = pltpu.async_copy(
            bufs.at[j % NBUF], out_hbm.at[bt, pl.ds(p0 + sub * CH, CH)], wsem)

    for j in sorted(writebacks):
        writebacks.pop(j).wait()


@jax.jit
def _embed(x_flat, tok_table, pos_table):
    run = pl.kernel(
        _body,
        out_type=jax.ShapeDtypeStruct((B, S, D), jnp.float32),
        mesh=plsc.VectorSubcoreMesh(
            core_axis_name="c", subcore_axis_name="s",
            num_cores=NC, num_subcores=NS,
        ),
        scratch_types=[
            pltpu.VMEM((NCH, CH), jnp.int32),
            pltpu.VMEM((PPW, D), jnp.float32),
            pltpu.VMEM((NBUF, CH, D), jnp.float32),
            pltpu.SemaphoreType.DMA,
            pltpu.SemaphoreType.DMA,
            pltpu.SemaphoreType.DMA,
        ],
    )
    return run(x_flat, tok_table, pos_table)


def kernel(x, tok_table, pos_table):
    return _embed(x.astype(jnp.int32), tok_table, pos_table)


# speedup vs baseline: 1.0191x; 1.0191x over previous
import jax
import jax.numpy as jnp
from jax import lax
from jax.experimental import pallas as pl
from jax.experimental.pallas import tpu as pltpu
from jax.experimental.pallas import tpu_sc as plsc

B = 4
S = 8192
D = 128
N = B * S

NC = 2
NS = 16
NW = NC * NS

PPW = S // NW
CH = 128
SUBS = PPW // CH
NCH = B * SUBS
NBUF = 5
LA = 4
LANES = 16
VPR = D // LANES


def _body(x_hbm, tok_hbm, pos_hbm, out_hbm, idx_v, pos_v, bufs, isem, gsem, wsem):
    c = lax.axis_index("c")
    s = lax.axis_index("s")
    wid = s * NC + c
    p0 = wid * PPW

    idx_copies = [
        pltpu.async_copy(
            x_hbm.at[bt, pl.ds(p0 + sub * CH, CH)],
            idx_v.at[bt * SUBS + sub],
            isem,
        )
        for bt in range(B)
        for sub in range(SUBS)
    ]
    pltpu.sync_copy(pos_hbm.at[pl.ds(p0, PPW)], pos_v)
    for cp in idx_copies:
        cp.wait()


    gathers, writebacks = {}, {}

    def prefill_and_gather(t):
        if t < NCH:
            if t >= NBUF:
                for w in writebacks.pop(t - NBUF):
                    w.wait()
            buf = bufs.at[t % NBUF]
            prow = (t % SUBS) * CH

            def copy_row(r, carry):
                for k in range(VPR):
                    sl = pl.ds(k * LANES, LANES)
                    buf[r, sl] = pos_v[prow + r, sl]
                return carry

            lax.fori_loop(0, CH, copy_row, 0)
            gathers[t] = pltpu.async_copy(
                tok_hbm.at[idx_v.at[t]], buf, gsem, add=True)

    for t in range(LA):
        prefill_and_gather(t)

    for j in range(NCH):
        prefill_and_gather(j + LA)
        gathers.pop(j).wait()
        bt, sub = j // SUBS, j % SUBS
        writebacks[j] = [
            pltpu.async_copy(
                bufs.at[j % NBUF, pl.ds(h * (CH // 2), CH // 2)],
                out_hbm.at[bt, pl.ds(p0 + sub * CH + h * (CH // 2), CH // 2)],
                wsem)
            for h in range(2)
        ]

    for j in sorted(writebacks):
        for w in writebacks.pop(j):
            w.wait()


@jax.jit
def _embed(x_flat, tok_table, pos_table):
    run = pl.kernel(
        _body,
        out_type=jax.ShapeDtypeStruct((B, S, D), jnp.float32),
        mesh=plsc.VectorSubcoreMesh(
            core_axis_name="c", subcore_axis_name="s",
            num_cores=NC, num_subcores=NS,
        ),
        scratch_types=[
            pltpu.VMEM((NCH, CH), jnp.int32),
            pltpu.VMEM((PPW, D), jnp.float32),
            pltpu.VMEM((NBUF, CH, D), jnp.float32),
            pltpu.SemaphoreType.DMA,
            pltpu.SemaphoreType.DMA,
            pltpu.SemaphoreType.DMA,
        ],
    )
    return run(x_flat, tok_table, pos_table)


def kernel(x, tok_table, pos_table):
    return _embed(x.astype(jnp.int32), tok_table, pos_table)


# speedup vs baseline: 1.0217x; 1.0025x over previous
import jax
import jax.numpy as jnp
from jax import lax
from jax.experimental import pallas as pl
from jax.experimental.pallas import tpu as pltpu
from jax.experimental.pallas import tpu_sc as plsc

B = 4
S = 8192
D = 128
N = B * S

NC = 2
NS = 16
NW = NC * NS

PPW = S // NW
CH = 128
SUBS = PPW // CH
NCH = B * SUBS
NBUF = 5
LA = 4
LANES = 16
VPR = D // LANES


def _body(x_hbm, tok_hbm, pos_hbm, out_hbm, idx_v, pos_v, bufs, isem, gsem, wsem):
    c = lax.axis_index("c")
    s = lax.axis_index("s")
    wid = s * NC + c
    p0 = wid * PPW

    idx_copies = [
        pltpu.async_copy(
            x_hbm.at[bt, pl.ds(p0 + sub * CH, CH)],
            idx_v.at[bt * SUBS + sub],
            isem,
        )
        for bt in range(B)
        for sub in range(SUBS)
    ]
    pltpu.sync_copy(pos_hbm.at[pl.ds(p0, PPW)], pos_v)
    for cp in idx_copies:
        cp.wait()


    gathers, writebacks = {}, {}

    def prefill_and_gather(t):
        if t < NCH:
            if t >= NBUF:
                writebacks.pop(t - NBUF).wait()
            buf = bufs.at[t % NBUF]
            prow = (t % SUBS) * CH

            def copy_row(r, carry):
                for k in range(VPR):
                    sl = pl.ds(k * LANES, LANES)
                    buf[r, sl] = pos_v[prow + r, sl]
                return carry

            lax.fori_loop(0, CH, copy_row, 0)
            gathers[t] = pltpu.async_copy(
                tok_hbm.at[idx_v.at[t]], buf, gsem, add=True)

    for t in range(LA):
        prefill_and_gather(t)

    for j in range(NCH):
        prefill_and_gather(j + LA)
        gathers.pop(j).wait()
        bt, sub = j // SUBS, j % SUBS
        writebacks[j] = pltpu.async_copy(
            bufs.at[j % NBUF], out_hbm.at[bt, pl.ds(p0 + sub * CH, CH)], wsem)

    for j in sorted(writebacks):
        writebacks.pop(j).wait()


@jax.jit
def _embed(x_flat, tok_table, pos_table):
    run = pl.kernel(
        _body,
        out_type=jax.ShapeDtypeStruct((B, S, D), jnp.float32),
        mesh=plsc.VectorSubcoreMesh(
            core_axis_name="c", subcore_axis_name="s",
            num_cores=NC, num_subcores=NS,
        ),
        scratch_types=[
            pltpu.VMEM((NCH, CH), jnp.int32),
            pltpu.VMEM((PPW, D), jnp.float32),
            pltpu.VMEM((NBUF, CH, D), jnp.float32),
            pltpu.SemaphoreType.DMA,
            pltpu.SemaphoreType.DMA,
            pltpu.SemaphoreType.DMA,
        ],
    )
    return run(x_flat, tok_table, pos_table)


def kernel(x, tok_table, pos_table):
    return _embed(x.astype(jnp.int32), tok_table, pos_table)
